# baseline (speedup 1.0000x reference)
import functools

import jax
import jax.numpy as jnp
from jax import lax
from jax.experimental import pallas as pl
from jax.experimental.pallas import tpu as pltpu
from jax.experimental.pallas import tpu_sc as plsc

_BATCH = 16384
_FIELDS = 26
_DIM = 32
_TOTAL = _BATCH * _FIELDS
_NC = 2
_NS = 16
_NW = _NC * _NS
_PER_W = _TOTAL // _NW
_BPW = _BATCH // _NW
_CB = 16
_CHUNK = _CB * _FIELDS
_NCHUNK = _BPW // _CB
_NBUF = 4

_mesh = plsc.VectorSubcoreMesh(
    core_axis_name="c", subcore_axis_name="s", num_cores=_NC, num_subcores=_NS
)


@functools.partial(
    pl.kernel,
    mesh=_mesh,
    out_type=jax.ShapeDtypeStruct((_BATCH, _FIELDS, _DIM), jnp.float32),
    scratch_types=[
        pltpu.VMEM((_NCHUNK, _CHUNK), jnp.int32),
        pltpu.VMEM((_NBUF, _CHUNK, _DIM), jnp.float32),
        pltpu.SemaphoreType.DMA,
        pltpu.SemaphoreType.DMA,
    ],
    compiler_params=pltpu.CompilerParams(use_tc_tiling_on_sc=False),
)
def _gather_kernel(table_hbm, idx_hbm, out_hbm, idx_v, rows_v, sem_g, sem_s):
    wid = lax.axis_index("s") * _NC + lax.axis_index("c")
    pltpu.sync_copy(idx_hbm.at[wid], idx_v)

    for j in range(_NBUF - 1):
        pltpu.async_copy(table_hbm.at[idx_v.at[j]], rows_v.at[j], sem_g)

    @pl.loop(0, _NCHUNK)
    def _chunk(j):
        b = lax.rem(j, _NBUF)
        pltpu.make_async_copy(
            table_hbm.at[idx_v.at[j]], rows_v.at[b], sem_g
        ).wait()

        row0 = wid * _BPW + j * _CB
        for k in range(_CB):
            pltpu.async_copy(
                rows_v.at[b, pl.ds(k * _FIELDS, _FIELDS)],
                out_hbm.at[row0 + k],
                sem_s,
            )

        @pl.when(j >= 1)
        def _():
            for k in range(_CB):
                pltpu.make_async_copy(
                    rows_v.at[0, pl.ds(0, _FIELDS)], out_hbm.at[0], sem_s
                ).wait()

        @pl.when(j + _NBUF - 1 < _NCHUNK)
        def _():
            nxt = j + _NBUF - 1
            pltpu.async_copy(
                table_hbm.at[idx_v.at[nxt]], rows_v.at[lax.rem(nxt, _NBUF)], sem_g
            )

    for k in range(_CB):
        pltpu.make_async_copy(
            rows_v.at[0, pl.ds(0, _FIELDS)], out_hbm.at[0], sem_s
        ).wait()


def kernel(x, table):
    idx = x.reshape(_NW, _NCHUNK, _CHUNK).astype(jnp.int32)
    return _gather_kernel(table, idx)


# speedup vs baseline: 1.0021x; 1.0021x over previous
import functools

import jax
import jax.numpy as jnp
from jax import lax
from jax.experimental import pallas as pl
from jax.experimental.pallas import tpu as pltpu
from jax.experimental.pallas import tpu_sc as plsc

_BATCH = 16384
_FIELDS = 26
_DIM = 32
_NC = 2
_NS = 16
_NW = _NC * _NS
_BPW = _BATCH // _NW
_NBUF = 4

_mesh = plsc.VectorSubcoreMesh(
    core_axis_name="c", subcore_axis_name="s", num_cores=_NC, num_subcores=_NS
)


@functools.partial(
    pl.kernel,
    mesh=_mesh,
    out_type=jax.ShapeDtypeStruct((_BATCH, _FIELDS, _DIM), jnp.float32),
    scratch_types=[
        pltpu.VMEM((_FIELDS, _BPW), jnp.int32),
        pltpu.VMEM((_NBUF, _BPW, _DIM), jnp.float32),
        pltpu.SemaphoreType.DMA,
    ],
    compiler_params=pltpu.CompilerParams(use_tc_tiling_on_sc=False),
)
def _gather_kernel(table_hbm, idxt_hbm, out_hbm, idx_v, rows_v, sem):
    wid = lax.axis_index("s") * _NC + lax.axis_index("c")
    b0 = wid * _BPW
    pltpu.sync_copy(idxt_hbm.at[:, pl.ds(b0, _BPW)], idx_v)

    for f in range(_NBUF - 1):
        pltpu.async_copy(table_hbm.at[idx_v.at[f]], rows_v.at[f], sem)

    @pl.loop(0, _FIELDS)
    def _field(f):
        b = lax.rem(f, _NBUF)
        pltpu.make_async_copy(table_hbm.at[idx_v.at[f]], rows_v.at[b], sem).wait()

        pltpu.sync_copy(rows_v.at[b], out_hbm.at[pl.ds(b0, _BPW), f])

        @pl.when(f + _NBUF - 1 < _FIELDS)
        def _():
            nxt = f + _NBUF - 1
            pltpu.async_copy(
                table_hbm.at[idx_v.at[nxt]], rows_v.at[lax.rem(nxt, _NBUF)], sem
            )


def kernel(x, table):
    return _gather_kernel(table, x.T.astype(jnp.int32))
